# baseline (speedup 1.0000x reference)
import functools

import jax
import jax.numpy as jnp
from jax import lax
from jax.experimental import pallas as pl
from jax.experimental.pallas import tpu as pltpu
from jax.experimental.pallas import tpu_sc as plsc

EMBED = 32
BATCH = 16384
SEQ = 50
VOCAB = 1000000

NC = 2
NS = 16
NW = NC * NS

VB = 1536
N_FULL = VOCAB // VB
TAIL = VOCAB - N_FULL * VB
BLOCKS_PER_W = -(-N_FULL // NW)

ROWS_PER_W = BATCH // NW
CHUNK = 64
N_CHUNKS = ROWS_PER_W // CHUNK
GROUP = 16
N_GROUPS = CHUNK // GROUP
INV_S = 1.0 / SEQ

_mesh = plsc.VectorSubcoreMesh(core_axis_name="c", subcore_axis_name="s")


@functools.partial(
    pl.kernel,
    mesh=_mesh,
    out_type=jax.ShapeDtypeStruct((VOCAB // 4, 4 * EMBED), jnp.float32),
    compiler_params=pltpu.CompilerParams(needs_layout_passes=False),
    scratch_types=[
        pltpu.VMEM((EMBED, VB), jnp.float32),
        pltpu.VMEM((VB // 4, 4 * EMBED), jnp.float32),
        pltpu.SemaphoreType.DMA,
    ],
)
def _transpose(wvt_hbm, tail_hbm, out_hbm, in_v, out_v, sem):
    wid = lax.axis_index("s") * NC + lax.axis_index("c")
    lo = lax.iota(jnp.int32, 16)
    hi = lo + 16

    def do_block(v0, nrows, width):
        v0 = pl.multiple_of(v0, 128)
        pltpu.sync_copy(wvt_hbm.at[:, pl.ds(v0, width)], in_v.at[:, pl.ds(0, width)])

        def row_body(j, carry):
            for q in range(4):
                v = j * 4 + q
                col = jnp.zeros((16,), jnp.int32) + v
                out_v[j, pl.ds(q * EMBED, 16)] = plsc.load_gather(in_v, [lo, col])
                out_v[j, pl.ds(q * EMBED + 16, 16)] = plsc.load_gather(in_v, [hi, col])
            return carry

        lax.fori_loop(0, nrows, row_body, 0)
        pltpu.sync_copy(
            out_v.at[pl.ds(0, nrows)],
            out_hbm.at[pl.ds(pl.multiple_of(v0 // 4, 8), nrows)],
        )

    def block_body(i, carry):
        b = i * NW + wid

        @pl.when(b < N_FULL)
        def _():
            do_block(b * VB, VB // 4, VB)

        return carry

    lax.fori_loop(0, BLOCKS_PER_W, block_body, 0)

    @pl.when(wid == 0)
    def _():
        pltpu.sync_copy(tail_hbm, out_hbm.at[pl.ds(N_FULL * VB // 4, TAIL // 4)])


@functools.partial(
    pl.kernel,
    mesh=_mesh,
    out_type=jax.ShapeDtypeStruct((BATCH, EMBED), jnp.float32),
    compiler_params=pltpu.CompilerParams(use_tc_tiling_on_sc=False),
    scratch_types=[
        pltpu.VMEM((CHUNK, SEQ), jnp.int32),
        pltpu.VMEM((CHUNK * SEQ, EMBED), jnp.float32),
        pltpu.VMEM((CHUNK, EMBED), jnp.float32),
        pltpu.SemaphoreType.DMA,
    ],
)
def _emb(idx_hbm, table_hbm, out_hbm, idx_v, rows_v, out_v, sem):
    wid = lax.axis_index("s") * NC + lax.axis_index("c")

    def fire(j):
        return pltpu.async_copy(
            table_hbm.at[idx_v.at[j]],
            rows_v.at[pl.ds(j * SEQ, SEQ)],
            sem,
        )

    def chunk_body(k, carry):
        chunk_id = wid * N_CHUNKS + k
        row0 = chunk_id * CHUNK
        pltpu.sync_copy(idx_hbm.at[pl.ds(row0, CHUNK)], idx_v)
        pending = [fire(j) for j in range(GROUP)]
        for g in range(1, N_GROUPS):
            nxt = [fire(g * GROUP + j) for j in range(GROUP)]
            for c in pending:
                c.wait()
            pending = nxt
        for c in pending:
            c.wait()

        def row_body(c, carry2):
            base = c * SEQ
            a0 = rows_v[base, pl.ds(0, 16)]
            a1 = rows_v[base, pl.ds(16, 16)]
            b0 = rows_v[base + 1, pl.ds(0, 16)]
            b1 = rows_v[base + 1, pl.ds(16, 16)]
            for s in range(2, SEQ, 2):
                a0 = a0 + rows_v[base + s, pl.ds(0, 16)]
                a1 = a1 + rows_v[base + s, pl.ds(16, 16)]
                b0 = b0 + rows_v[base + s + 1, pl.ds(0, 16)]
                b1 = b1 + rows_v[base + s + 1, pl.ds(16, 16)]
            out_v[c, pl.ds(0, 16)] = (a0 + b0) * INV_S
            out_v[c, pl.ds(16, 16)] = (a1 + b1) * INV_S
            return carry2

        lax.fori_loop(0, CHUNK, row_body, 0)
        pltpu.sync_copy(out_v, out_hbm.at[pl.ds(row0, CHUNK)])
        return carry

    lax.fori_loop(0, N_CHUNKS, chunk_body, 0)


def kernel(input, word_vectors):
    tail_q = word_vectors[N_FULL * VB :].reshape(TAIL // 4, 4 * EMBED)
    packed = _transpose(word_vectors.T, tail_q)
    table = packed.reshape(VOCAB, EMBED)
    return _emb(input.astype(jnp.int32), table)


# speedup vs baseline: 1.0004x; 1.0004x over previous
import functools

import jax
import jax.numpy as jnp
from jax import lax
from jax.experimental import pallas as pl
from jax.experimental.pallas import tpu as pltpu
from jax.experimental.pallas import tpu_sc as plsc

EMBED = 32
BATCH = 16384
SEQ = 50
VOCAB = 1000000

NC = 2
NS = 16
NW = NC * NS

VB = 1536
N_FULL = VOCAB // VB
TAIL = VOCAB - N_FULL * VB
BLOCKS_PER_W = -(-N_FULL // NW)

ROWS_PER_W = BATCH // NW
CHUNK = 64
N_CHUNKS = ROWS_PER_W // CHUNK
GROUP = 16
N_GROUPS = CHUNK // GROUP
INV_S = 1.0 / SEQ

_mesh = plsc.VectorSubcoreMesh(core_axis_name="c", subcore_axis_name="s")


@functools.partial(
    pl.kernel,
    mesh=_mesh,
    out_type=jax.ShapeDtypeStruct((VOCAB // 4, 4 * EMBED), jnp.float32),
    compiler_params=pltpu.CompilerParams(needs_layout_passes=False),
    scratch_types=[
        pltpu.VMEM((EMBED, VB + 1), jnp.float32),
        pltpu.VMEM((VB // 4, 4 * EMBED), jnp.float32),
        pltpu.SemaphoreType.DMA,
    ],
)
def _transpose(wvt_hbm, tail_hbm, out_hbm, in_v, out_v, sem):
    wid = lax.axis_index("s") * NC + lax.axis_index("c")
    lo = lax.iota(jnp.int32, 16)
    hi = lo + 16

    def do_block(v0, nrows, width):
        v0 = pl.multiple_of(v0, 128)
        pltpu.sync_copy(wvt_hbm.at[:, pl.ds(v0, width)], in_v.at[:, pl.ds(0, width)])

        def row_body(j, carry):
            for q in range(4):
                v = j * 4 + q
                col = jnp.zeros((16,), jnp.int32) + v
                out_v[j, pl.ds(q * EMBED, 16)] = plsc.load_gather(in_v, [lo, col])
                out_v[j, pl.ds(q * EMBED + 16, 16)] = plsc.load_gather(in_v, [hi, col])
            return carry

        lax.fori_loop(0, nrows, row_body, 0)
        pltpu.sync_copy(
            out_v.at[pl.ds(0, nrows)],
            out_hbm.at[pl.ds(pl.multiple_of(v0 // 4, 8), nrows)],
        )

    def block_body(i, carry):
        b = i * NW + wid

        @pl.when(b < N_FULL)
        def _():
            do_block(b * VB, VB // 4, VB)

        return carry

    lax.fori_loop(0, BLOCKS_PER_W, block_body, 0)

    @pl.when(wid == 0)
    def _():
        pltpu.sync_copy(tail_hbm, out_hbm.at[pl.ds(N_FULL * VB // 4, TAIL // 4)])


@functools.partial(
    pl.kernel,
    mesh=_mesh,
    out_type=jax.ShapeDtypeStruct((BATCH, EMBED), jnp.float32),
    compiler_params=pltpu.CompilerParams(use_tc_tiling_on_sc=False),
    scratch_types=[
        pltpu.VMEM((CHUNK, SEQ), jnp.int32),
        pltpu.VMEM((CHUNK * SEQ, EMBED), jnp.float32),
        pltpu.VMEM((CHUNK, EMBED), jnp.float32),
        pltpu.SemaphoreType.DMA,
    ],
)
def _emb(idx_hbm, table_hbm, out_hbm, idx_v, rows_v, out_v, sem):
    wid = lax.axis_index("s") * NC + lax.axis_index("c")

    def fire(j):
        return pltpu.async_copy(
            table_hbm.at[idx_v.at[j]],
            rows_v.at[pl.ds(j * SEQ, SEQ)],
            sem,
        )

    def chunk_body(k, carry):
        chunk_id = wid * N_CHUNKS + k
        row0 = chunk_id * CHUNK
        pltpu.sync_copy(idx_hbm.at[pl.ds(row0, CHUNK)], idx_v)
        pending = [fire(j) for j in range(GROUP)]
        for g in range(1, N_GROUPS):
            nxt = [fire(g * GROUP + j) for j in range(GROUP)]
            for c in pending:
                c.wait()
            pending = nxt
        for c in pending:
            c.wait()

        def row_body(c, carry2):
            base = c * SEQ
            a0 = rows_v[base, pl.ds(0, 16)]
            a1 = rows_v[base, pl.ds(16, 16)]
            b0 = rows_v[base + 1, pl.ds(0, 16)]
            b1 = rows_v[base + 1, pl.ds(16, 16)]
            for s in range(2, SEQ, 2):
                a0 = a0 + rows_v[base + s, pl.ds(0, 16)]
                a1 = a1 + rows_v[base + s, pl.ds(16, 16)]
                b0 = b0 + rows_v[base + s + 1, pl.ds(0, 16)]
                b1 = b1 + rows_v[base + s + 1, pl.ds(16, 16)]
            out_v[c, pl.ds(0, 16)] = (a0 + b0) * INV_S
            out_v[c, pl.ds(16, 16)] = (a1 + b1) * INV_S
            return carry2

        lax.fori_loop(0, CHUNK, row_body, 0)
        pltpu.sync_copy(out_v, out_hbm.at[pl.ds(row0, CHUNK)])
        return carry

    lax.fori_loop(0, N_CHUNKS, chunk_body, 0)


def kernel(input, word_vectors):
    tail_q = word_vectors[N_FULL * VB :].reshape(TAIL // 4, 4 * EMBED)
    packed = _transpose(word_vectors.T, tail_q)
    table = packed.reshape(VOCAB, EMBED)
    return _emb(input.astype(jnp.int32), table)


# speedup vs baseline: 1.6704x; 1.6696x over previous
import functools

import jax
import jax.numpy as jnp
from jax import lax
from jax.experimental import pallas as pl
from jax.experimental.pallas import tpu as pltpu
from jax.experimental.pallas import tpu_sc as plsc

EMBED = 32
BATCH = 16384
SEQ = 50
VOCAB = 1000000

NC = 2
NS = 16
NW = NC * NS

ROWS_PER_W = BATCH // NW
CHUNK = 64
N_CHUNKS = ROWS_PER_W // CHUNK
GROUP = 16
N_GROUPS = CHUNK // GROUP
INV_S = 1.0 / SEQ

_mesh = plsc.VectorSubcoreMesh(core_axis_name="c", subcore_axis_name="s")


@functools.partial(
    pl.kernel,
    mesh=_mesh,
    out_type=jax.ShapeDtypeStruct((BATCH, EMBED), jnp.float32),
    compiler_params=pltpu.CompilerParams(use_tc_tiling_on_sc=False),
    scratch_types=[
        pltpu.VMEM((CHUNK, SEQ), jnp.int32),
        pltpu.VMEM((CHUNK * SEQ, EMBED), jnp.float32),
        pltpu.VMEM((CHUNK, EMBED), jnp.float32),
        pltpu.SemaphoreType.DMA,
    ],
)
def _emb(idx_hbm, table_hbm, out_hbm, idx_v, rows_v, out_v, sem):
    wid = lax.axis_index("s") * NC + lax.axis_index("c")

    def fire(j):
        return pltpu.async_copy(
            table_hbm.at[idx_v.at[j]],
            rows_v.at[pl.ds(j * SEQ, SEQ)],
            sem,
        )

    def chunk_body(k, carry):
        chunk_id = wid * N_CHUNKS + k
        row0 = chunk_id * CHUNK
        pltpu.sync_copy(idx_hbm.at[pl.ds(row0, CHUNK)], idx_v)
        pending = [fire(j) for j in range(GROUP)]
        for g in range(1, N_GROUPS):
            nxt = [fire(g * GROUP + j) for j in range(GROUP)]
            for c in pending:
                c.wait()
            pending = nxt
        for c in pending:
            c.wait()

        def row_body(c, carry2):
            base = c * SEQ
            a0 = rows_v[base, pl.ds(0, 16)]
            a1 = rows_v[base, pl.ds(16, 16)]
            b0 = rows_v[base + 1, pl.ds(0, 16)]
            b1 = rows_v[base + 1, pl.ds(16, 16)]
            for s in range(2, SEQ, 2):
                a0 = a0 + rows_v[base + s, pl.ds(0, 16)]
                a1 = a1 + rows_v[base + s, pl.ds(16, 16)]
                b0 = b0 + rows_v[base + s + 1, pl.ds(0, 16)]
                b1 = b1 + rows_v[base + s + 1, pl.ds(16, 16)]
            out_v[c, pl.ds(0, 16)] = (a0 + b0) * INV_S
            out_v[c, pl.ds(16, 16)] = (a1 + b1) * INV_S
            return carry2

        lax.fori_loop(0, CHUNK, row_body, 0)
        pltpu.sync_copy(out_v, out_hbm.at[pl.ds(row0, CHUNK)])
        return carry

    lax.fori_loop(0, N_CHUNKS, chunk_body, 0)


def kernel(input, word_vectors):
    wv128 = jnp.pad(word_vectors, ((0, 0), (0, 3 * EMBED)))
    table4 = wv128.reshape(4 * VOCAB, EMBED)
    return _emb(input.astype(jnp.int32) * 4, table4)
